# baseline (speedup 1.0000x reference)
import numpy as np
import jax
import jax.numpy as jnp
from jax import lax
from jax.experimental import pallas as pl
from jax.experimental.pallas import tpu as pltpu

_VLIM = 48 * 1024 * 1024
_EPS = 1e-5


def _band_weight(w9, cin, cout, W, kpad, ngroups):
    w = w9.reshape(3, 3, cin, cout)
    t = np.zeros((3, W, W), np.float32)
    for kw in range(3):
        for wo in range(W):
            wi = wo + kw - 1
            if 0 <= wi < W:
                t[kw, wi, wo] = 1.0
    b = jnp.einsum("kqio,qwv->kwivo", w, jnp.asarray(t))
    b = b.reshape(3, W * cin, W, cout)
    if kpad > W * cin:
        b = jnp.pad(b, ((0, 0), (0, kpad - W * cin), (0, 0), (0, 0)))
    be = b[:, :, 0::2].reshape(3, kpad, -1)
    bo = b[:, :, 1::2].reshape(3, kpad, -1)
    ge, go = ngroups
    be = jnp.pad(be, ((0, 0), (0, 0), (0, ge - be.shape[-1])))
    bo = jnp.pad(bo, ((0, 0), (0, 0), (0, go - bo.shape[-1])))
    return jnp.concatenate([be, bo], axis=-1).reshape(
        3 * kpad, ge + go).astype(jnp.bfloat16)


def _bn_relu(p_ref, s_ref, t_ref, half, off):
    s = s_ref[...].reshape(1, 1, half)
    t = t_ref[...].reshape(1, 1, half)
    hi = s * p_ref[:, :, 0:half].astype(jnp.float32) + t
    lo = s * p_ref[:, :, off:off + half].astype(jnp.float32) + t
    return jnp.maximum(jnp.where(s > 0, hi, lo), 0.0)


def _pair_stats(vmax, vmin, st_ref):
    st_ref[0, 0:1, :] = (jnp.sum(vmax, 0, keepdims=True)
                         + jnp.sum(vmin, 0, keepdims=True))
    st_ref[0, 1:2, :] = (jnp.sum(vmax * vmax, 0, keepdims=True)
                         + jnp.sum(vmin * vmin, 0, keepdims=True))


def _k1(x_ref, w_ref, o_ref, st_ref, y_ref):
    bt = x_ref.shape[0]
    z = jnp.zeros((bt, 1, 28), jnp.float32)
    v = jnp.concatenate([z, x_ref[...], z], axis=1).reshape(bt, 15, 2, 28)
    le = jnp.concatenate([v[:, 0:14, 0], v[:, 0:14, 1], v[:, 1:15, 0]],
                         axis=2)
    lo = jnp.concatenate([v[:, 0:14, 1], v[:, 1:15, 0], v[:, 1:15, 1]],
                         axis=2)
    lhs = jnp.concatenate([le.reshape(bt * 14, 84),
                           lo.reshape(bt * 14, 84)], axis=0)
    y_ref[...] = jnp.dot(lhs.astype(jnp.bfloat16), w_ref[...],
                         preferred_element_type=jnp.float32)
    m = bt * 14
    vmax = jnp.maximum(y_ref[0:m, :], y_ref[m:2 * m, :])
    vmin = jnp.minimum(y_ref[0:m, :], y_ref[m:2 * m, :])
    _pair_stats(vmax, vmin, st_ref)
    pmax = jnp.maximum(vmax[:, 0:448], vmax[:, 512:960])
    pmin = jnp.minimum(vmin[:, 0:448], vmin[:, 512:960])
    o_ref[:, :, 0:448] = pmax.reshape(bt, 14, 448).astype(jnp.bfloat16)
    o_ref[:, :, 512:960] = pmin.reshape(bt, 14, 448).astype(jnp.bfloat16)


def _k2(p_ref, s_ref, t_ref, w_ref, o_ref, st_ref, y_ref):
    bt = p_ref.shape[0]
    a = _bn_relu(p_ref, s_ref, t_ref, 448, 512)
    a = jnp.pad(a, ((0, 0), (1, 1), (0, 64)))
    v = a.reshape(bt, 8, 2, 512)
    le = jnp.concatenate([v[:, 0:7, 0], v[:, 0:7, 1], v[:, 1:8, 0]], axis=2)
    lo = jnp.concatenate([v[:, 0:7, 1], v[:, 1:8, 0], v[:, 1:8, 1]], axis=2)
    lhs = jnp.concatenate([le.reshape(bt * 7, 1536),
                           lo.reshape(bt * 7, 1536)], axis=0)
    y_ref[...] = jnp.dot(lhs.astype(jnp.bfloat16), w_ref[...],
                         preferred_element_type=jnp.float32)
    m = bt * 7
    vmax = jnp.maximum(y_ref[0:m, :], y_ref[m:2 * m, :])
    vmin = jnp.minimum(y_ref[0:m, :], y_ref[m:2 * m, :])
    _pair_stats(vmax, vmin, st_ref)
    pmax = jnp.maximum(vmax[:, 0:448], vmax[:, 512:960])
    pmin = jnp.minimum(vmin[:, 0:448], vmin[:, 512:960])
    o_ref[:, :, 0:448] = pmax.reshape(bt, 7, 448).astype(jnp.bfloat16)
    o_ref[:, :, 512:960] = pmin.reshape(bt, 7, 448).astype(jnp.bfloat16)


def _k3(p_ref, s_ref, t_ref, w_ref, o_ref, st_ref, y_ref):
    bt = p_ref.shape[0]
    a = _bn_relu(p_ref, s_ref, t_ref, 448, 512)
    a = jnp.pad(a, ((0, 0), (1, 2), (0, 64)))
    v = a.reshape(bt, 5, 2, 512)
    le = jnp.concatenate([v[:, 0:3, 0], v[:, 0:3, 1], v[:, 1:4, 0]], axis=2)
    lo = jnp.concatenate([v[:, 0:3, 1], v[:, 1:4, 0], v[:, 1:4, 1]], axis=2)
    lx = jnp.concatenate([v[:, 3:4, 0], v[:, 3:4, 1], v[:, 4:5, 0]], axis=2)
    lhs = jnp.concatenate([le.reshape(bt * 3, 1536),
                           lo.reshape(bt * 3, 1536),
                           lx.reshape(bt, 1536)], axis=0)
    y_ref[...] = jnp.dot(lhs.astype(jnp.bfloat16), w_ref[...],
                         preferred_element_type=jnp.float32)
    m = bt * 3
    vmax = jnp.maximum(y_ref[0:m, :], y_ref[m:2 * m, :])
    vmin = jnp.minimum(y_ref[0:m, :], y_ref[m:2 * m, :])
    yx = y_ref[2 * m:2 * m + bt, :]
    st_ref[0, 0:1, :] = (jnp.sum(vmax, 0, keepdims=True)
                         + jnp.sum(vmin, 0, keepdims=True)
                         + jnp.sum(yx, 0, keepdims=True))
    st_ref[0, 1:2, :] = (jnp.sum(vmax * vmax, 0, keepdims=True)
                         + jnp.sum(vmin * vmin, 0, keepdims=True)
                         + jnp.sum(yx * yx, 0, keepdims=True))
    pmax = jnp.maximum(vmax[:, 0:384], vmax[:, 512:896])
    pmin = jnp.minimum(vmin[:, 0:384], vmin[:, 512:896])
    o_ref[:, :, 0:384] = pmax.reshape(bt, 3, 384).astype(jnp.bfloat16)
    o_ref[:, :, 384:768] = pmin.reshape(bt, 3, 384).astype(jnp.bfloat16)


def _k4(p_ref, s_ref, t_ref, w1_ref, b1_ref, w2_ref, b2_ref, o_ref):
    bt = p_ref.shape[0]
    a = _bn_relu(p_ref, s_ref, t_ref, 384, 384)
    xf = a.reshape(bt, 1152)
    h = jnp.dot(xf, w1_ref[...], preferred_element_type=jnp.float32)
    h = jnp.maximum(h + b1_ref[...], 0.0)
    z = jnp.dot(h, w2_ref[...], preferred_element_type=jnp.float32)
    z = z + b2_ref[...]
    z = z - jnp.max(z, axis=-1, keepdims=True)
    o_ref[...] = z - jnp.log(jnp.sum(jnp.exp(z), axis=-1, keepdims=True))


def _cparams():
    return pltpu.CompilerParams(
        dimension_semantics=("parallel",), vmem_limit_bytes=_VLIM)


def _conv_stage(kern, p, s, t, w, *, bt, hin, lin, hout, lout, nlanes,
                yrows):
    n = p.shape[0]
    grid = (n // bt,)
    in_specs = [pl.BlockSpec((bt, hin, lin), lambda i: (i, 0, 0))]
    args = [p]
    if s is not None:
        in_specs += [pl.BlockSpec(s.shape, lambda i: (0, 0)),
                     pl.BlockSpec(t.shape, lambda i: (0, 0))]
        args += [s, t]
    in_specs.append(pl.BlockSpec(w.shape, lambda i: (0, 0)))
    args.append(w)
    return pl.pallas_call(
        kern,
        grid=grid,
        in_specs=in_specs,
        out_specs=[
            pl.BlockSpec((bt, hout, lout), lambda i: (i, 0, 0)),
            pl.BlockSpec((1, 2, nlanes), lambda i: (i, 0, 0)),
        ],
        out_shape=(
            jax.ShapeDtypeStruct((n, hout, lout), jnp.bfloat16),
            jax.ShapeDtypeStruct((grid[0], 2, nlanes), jnp.float32),
        ),
        scratch_shapes=[pltpu.VMEM((bt * yrows, nlanes), jnp.float32)],
        compiler_params=_cparams(),
    )(*args)


def _bn_pair(su, sq, gamma, beta, bias, cnt, reps):
    mean_nb = su / cnt
    var = sq / cnt - mean_nb * mean_nb
    scale = gamma * lax.rsqrt(var + _EPS)
    shift = beta - (mean_nb + bias) * scale
    half = reps * scale.shape[0]
    return (jnp.tile(scale, reps).reshape(1, half),
            jnp.tile(shift, reps).reshape(1, half))


def _pick_bt(n, cap):
    bt = cap
    while bt > 1 and n % bt:
        bt //= 2
    return bt


def kernel(x_nchw, conv1_w9, conv1_b, bn1_gamma, bn1_beta,
           conv2_w9, conv2_b, bn2_gamma, bn2_beta,
           conv3_w9, conv3_b, bn3_gamma, bn3_beta,
           fc1_w_t, fc1_b, fc2_w_t, fc2_b):
    n = x_nchw.shape[0]
    x = x_nchw.reshape(n, 28, 28).astype(jnp.float32)

    bw1 = _band_weight(conv1_w9, 1, 32, 28, 28, (512, 512))
    bw2 = _band_weight(conv2_w9, 32, 64, 14, 512, (512, 512))
    bw3 = _band_weight(conv3_w9, 64, 128, 7, 512, (512, 384))

    bt1 = _pick_bt(n, 128)
    p1, st1 = _conv_stage(_k1, x, None, None, bw1, bt=bt1, hin=28, lin=28,
                          hout=14, lout=1024, nlanes=1024, yrows=28)
    tot1 = jnp.sum(st1, axis=0)
    ch1 = (tot1[:, 0:448].reshape(2, 14, 32)
           + tot1[:, 512:960].reshape(2, 14, 32)).sum(axis=1)
    s1, t1 = _bn_pair(ch1[0], ch1[1], bn1_gamma, bn1_beta, conv1_b,
                      jnp.float32(n * 784), 14)

    bt2 = _pick_bt(n, 128)
    p2, st2 = _conv_stage(_k2, p1, s1, t1, bw2, bt=bt2, hin=14, lin=1024,
                          hout=7, lout=1024, nlanes=1024, yrows=14)
    tot2 = jnp.sum(st2, axis=0)
    ch2 = (tot2[:, 0:448].reshape(2, 7, 64)
           + tot2[:, 512:960].reshape(2, 7, 64)).sum(axis=1)
    s2, t2 = _bn_pair(ch2[0], ch2[1], bn2_gamma, bn2_beta, conv2_b,
                      jnp.float32(n * 196), 7)

    bt3 = _pick_bt(n, 256)
    p3, st3 = _conv_stage(_k3, p2, s2, t2, bw3, bt=bt3, hin=7, lin=1024,
                          hout=3, lout=768, nlanes=896, yrows=7)
    tot3 = jnp.sum(st3, axis=0)
    ch3 = (tot3[:, 0:512].reshape(2, 4, 128).sum(axis=1)
           + tot3[:, 512:896].reshape(2, 3, 128).sum(axis=1))
    s3, t3 = _bn_pair(ch3[0], ch3[1], bn3_gamma, bn3_beta, conv3_b,
                      jnp.float32(n * 49), 3)

    bt4 = _pick_bt(n, 512)
    out = pl.pallas_call(
        _k4,
        grid=(n // bt4,),
        in_specs=[
            pl.BlockSpec((bt4, 3, 768), lambda i: (i, 0, 0)),
            pl.BlockSpec((1, 384), lambda i: (0, 0)),
            pl.BlockSpec((1, 384), lambda i: (0, 0)),
            pl.BlockSpec(fc1_w_t.shape, lambda i: (0, 0)),
            pl.BlockSpec((1, 256), lambda i: (0, 0)),
            pl.BlockSpec(fc2_w_t.shape, lambda i: (0, 0)),
            pl.BlockSpec((1, 10), lambda i: (0, 0)),
        ],
        out_specs=pl.BlockSpec((bt4, 10), lambda i: (i, 0)),
        out_shape=jax.ShapeDtypeStruct((n, 10), jnp.float32),
        compiler_params=_cparams(),
    )(p3, s3, t3, fc1_w_t, fc1_b.reshape(1, 256), fc2_w_t,
      fc2_b.reshape(1, 10))
    return out


# speedup vs baseline: 1.3115x; 1.3115x over previous
import numpy as np
import jax
import jax.numpy as jnp
from jax import lax
from jax.experimental import pallas as pl
from jax.experimental.pallas import tpu as pltpu

_VLIM = 48 * 1024 * 1024
_EPS = 1e-5


def _band_weight(w9, cin, cout, W, kpad, ngroups):
    w = w9.reshape(3, 3, cin, cout)
    t = np.zeros((3, W, W), np.float32)
    for kw in range(3):
        for wo in range(W):
            wi = wo + kw - 1
            if 0 <= wi < W:
                t[kw, wi, wo] = 1.0
    b = jnp.einsum("kqio,qwv->kwivo", w, jnp.asarray(t))
    b = b.reshape(3, W * cin, W, cout)
    if kpad > W * cin:
        b = jnp.pad(b, ((0, 0), (0, kpad - W * cin), (0, 0), (0, 0)))
    be = b[:, :, 0::2].reshape(3, kpad, -1)
    bo = b[:, :, 1::2].reshape(3, kpad, -1)
    ge, go = ngroups
    be = jnp.pad(be, ((0, 0), (0, 0), (0, ge - be.shape[-1])))
    bo = jnp.pad(bo, ((0, 0), (0, 0), (0, go - bo.shape[-1])))
    return jnp.concatenate([be, bo], axis=-1).reshape(
        3 * kpad, ge + go).astype(jnp.bfloat16)


def _bn_relu(p_ref, s_ref, t_ref, half, off):
    s = s_ref[...].reshape(1, 1, half)
    t = t_ref[...].reshape(1, 1, half)
    hi = s * p_ref[:, :, 0:half].astype(jnp.float32) + t
    lo = s * p_ref[:, :, off:off + half].astype(jnp.float32) + t
    return jnp.maximum(jnp.where(s > 0, hi, lo), 0.0)


def _pair_stats(vmax, vmin, st_ref):
    st_ref[0, 0:1, :] = (jnp.sum(vmax, 0, keepdims=True)
                         + jnp.sum(vmin, 0, keepdims=True))
    st_ref[0, 1:2, :] = (jnp.sum(vmax * vmax, 0, keepdims=True)
                         + jnp.sum(vmin * vmin, 0, keepdims=True))


def _k1(x_ref, w_ref, o_ref, st_ref, y_ref):
    bt = x_ref.shape[0]

    def band(j, i0):
        return x_ref[:, j, i0:i0 + 7, :]

    def half(spec):
        cols = [jnp.concatenate([band(*o), band(*e)], axis=1)
                for o, e in spec]
        return jnp.concatenate(cols, axis=2).reshape(bt * 14, 84)

    le = half([((2, 0), (0, 0)), ((3, 0), (1, 0)), ((0, 1), (2, 0))])
    lo = half([((3, 0), (1, 0)), ((0, 1), (2, 0)), ((1, 1), (3, 0))])
    lhs = jnp.concatenate([le, lo], axis=0)
    y_ref[...] = jnp.dot(lhs.astype(jnp.bfloat16), w_ref[...],
                         preferred_element_type=jnp.float32)
    m = bt * 14
    vmax = jnp.maximum(y_ref[0:m, :], y_ref[m:2 * m, :])
    vmin = jnp.minimum(y_ref[0:m, :], y_ref[m:2 * m, :])
    _pair_stats(vmax, vmin, st_ref)
    pmax = jnp.maximum(vmax[:, 0:448], vmax[:, 512:960])
    pmin = jnp.minimum(vmin[:, 0:448], vmin[:, 512:960])
    o_ref[:, :, 0:448] = pmax.reshape(bt, 14, 448).astype(jnp.bfloat16)
    o_ref[:, :, 512:960] = pmin.reshape(bt, 14, 448).astype(jnp.bfloat16)


def _k2(p_ref, s_ref, t_ref, w_ref, o_ref, st_ref, y_ref):
    bt = p_ref.shape[0]
    a = _bn_relu(p_ref, s_ref, t_ref, 448, 512)
    a = jnp.pad(a, ((0, 0), (1, 1), (0, 64)))
    le = jnp.concatenate([a[:, 0:7], a[:, 8:15], a[:, 1:8]], axis=2)
    lo = jnp.concatenate([a[:, 8:15], a[:, 1:8], a[:, 9:16]], axis=2)
    lhs = jnp.concatenate([le.reshape(bt * 7, 1536),
                           lo.reshape(bt * 7, 1536)], axis=0)
    y_ref[...] = jnp.dot(lhs.astype(jnp.bfloat16), w_ref[...],
                         preferred_element_type=jnp.float32)
    m = bt * 7
    vmax = jnp.maximum(y_ref[0:m, :], y_ref[m:2 * m, :])
    vmin = jnp.minimum(y_ref[0:m, :], y_ref[m:2 * m, :])
    _pair_stats(vmax, vmin, st_ref)
    pmax = jnp.maximum(vmax[:, 0:448], vmax[:, 512:960])
    pmin = jnp.minimum(vmin[:, 0:448], vmin[:, 512:960])
    o_ref[:, :, 0:448] = pmax.reshape(bt, 7, 448).astype(jnp.bfloat16)
    o_ref[:, :, 512:960] = pmin.reshape(bt, 7, 448).astype(jnp.bfloat16)


def _k3(p_ref, s_ref, t_ref, w_ref, o_ref, st_ref, y_ref):
    bt = p_ref.shape[0]
    a = _bn_relu(p_ref, s_ref, t_ref, 448, 512)
    a = jnp.pad(a, ((0, 0), (1, 2), (0, 64)))
    v = a.reshape(bt, 5, 2, 512)
    le = jnp.concatenate([v[:, 0:3, 0], v[:, 0:3, 1], v[:, 1:4, 0]], axis=2)
    lo = jnp.concatenate([v[:, 0:3, 1], v[:, 1:4, 0], v[:, 1:4, 1]], axis=2)
    lx = jnp.concatenate([v[:, 3:4, 0], v[:, 3:4, 1], v[:, 4:5, 0]], axis=2)
    lhs = jnp.concatenate([le.reshape(bt * 3, 1536),
                           lo.reshape(bt * 3, 1536),
                           lx.reshape(bt, 1536)], axis=0)
    y_ref[...] = jnp.dot(lhs.astype(jnp.bfloat16), w_ref[...],
                         preferred_element_type=jnp.float32)
    m = bt * 3
    vmax = jnp.maximum(y_ref[0:m, :], y_ref[m:2 * m, :])
    vmin = jnp.minimum(y_ref[0:m, :], y_ref[m:2 * m, :])
    yx = y_ref[2 * m:2 * m + bt, :]
    st_ref[0, 0:1, :] = (jnp.sum(vmax, 0, keepdims=True)
                         + jnp.sum(vmin, 0, keepdims=True)
                         + jnp.sum(yx, 0, keepdims=True))
    st_ref[0, 1:2, :] = (jnp.sum(vmax * vmax, 0, keepdims=True)
                         + jnp.sum(vmin * vmin, 0, keepdims=True)
                         + jnp.sum(yx * yx, 0, keepdims=True))
    pmax = jnp.maximum(vmax[:, 0:384], vmax[:, 512:896])
    pmin = jnp.minimum(vmin[:, 0:384], vmin[:, 512:896])
    o_ref[:, :, 0:384] = pmax.reshape(bt, 3, 384).astype(jnp.bfloat16)
    o_ref[:, :, 384:768] = pmin.reshape(bt, 3, 384).astype(jnp.bfloat16)


def _k4(p_ref, s_ref, t_ref, w1_ref, b1_ref, w2_ref, b2_ref, o_ref):
    bt = p_ref.shape[0]
    a = _bn_relu(p_ref, s_ref, t_ref, 384, 384)
    xf = a.reshape(bt, 1152)
    h = jnp.dot(xf, w1_ref[...], preferred_element_type=jnp.float32)
    h = jnp.maximum(h + b1_ref[...], 0.0)
    z = jnp.dot(h, w2_ref[...], preferred_element_type=jnp.float32)
    z = z + b2_ref[...]
    z = z - jnp.max(z, axis=-1, keepdims=True)
    o_ref[...] = z - jnp.log(jnp.sum(jnp.exp(z), axis=-1, keepdims=True))


def _cparams():
    return pltpu.CompilerParams(
        dimension_semantics=("parallel",), vmem_limit_bytes=_VLIM)


def _conv_stage(kern, p, s, t, w, *, bt, hout, lout, nlanes, yrows):
    n = p.shape[0]
    grid = (n // bt,)
    pblk = (bt,) + p.shape[1:]
    pmap = lambda i: (i,) + (0,) * (len(pblk) - 1)
    in_specs = [pl.BlockSpec(pblk, pmap)]
    args = [p]
    if s is not None:
        in_specs += [pl.BlockSpec(s.shape, lambda i: (0, 0)),
                     pl.BlockSpec(t.shape, lambda i: (0, 0))]
        args += [s, t]
    in_specs.append(pl.BlockSpec(w.shape, lambda i: (0, 0)))
    args.append(w)
    return pl.pallas_call(
        kern,
        grid=grid,
        in_specs=in_specs,
        out_specs=[
            pl.BlockSpec((bt, hout, lout), lambda i: (i, 0, 0)),
            pl.BlockSpec((1, 2, nlanes), lambda i: (i, 0, 0)),
        ],
        out_shape=(
            jax.ShapeDtypeStruct((n, hout, lout), jnp.bfloat16),
            jax.ShapeDtypeStruct((grid[0], 2, nlanes), jnp.float32),
        ),
        scratch_shapes=[pltpu.VMEM((bt * yrows, nlanes), jnp.float32)],
        compiler_params=_cparams(),
    )(*args)


def _bn_pair(su, sq, gamma, beta, bias, cnt, reps):
    mean_nb = su / cnt
    var = sq / cnt - mean_nb * mean_nb
    scale = gamma * lax.rsqrt(var + _EPS)
    shift = beta - (mean_nb + bias) * scale
    half = reps * scale.shape[0]
    return (jnp.tile(scale, reps).reshape(1, half),
            jnp.tile(shift, reps).reshape(1, half))


def _pick_bt(n, cap):
    bt = cap
    while bt > 1 and n % bt:
        bt //= 2
    return bt


def kernel(x_nchw, conv1_w9, conv1_b, bn1_gamma, bn1_beta,
           conv2_w9, conv2_b, bn2_gamma, bn2_beta,
           conv3_w9, conv3_b, bn3_gamma, bn3_beta,
           fc1_w_t, fc1_b, fc2_w_t, fc2_b):
    n = x_nchw.shape[0]
    x = x_nchw.reshape(n, 28, 28).astype(jnp.float32)
    xb = jnp.pad(x, ((0, 0), (1, 3), (0, 0))).reshape(n, 8, 4, 28)
    xb = jnp.transpose(xb, (0, 2, 1, 3))

    bw1 = _band_weight(conv1_w9, 1, 32, 28, 28, (512, 512))
    bw2 = _band_weight(conv2_w9, 32, 64, 14, 512, (512, 512))
    bw3 = _band_weight(conv3_w9, 64, 128, 7, 512, (512, 384))

    bt1 = _pick_bt(n, 64)
    p1, st1 = _conv_stage(_k1, xb, None, None, bw1, bt=bt1,
                          hout=14, lout=1024, nlanes=1024, yrows=28)
    tot1 = jnp.sum(st1, axis=0)
    ch1 = (tot1[:, 0:448].reshape(2, 14, 32)
           + tot1[:, 512:960].reshape(2, 14, 32)).sum(axis=1)
    s1, t1 = _bn_pair(ch1[0], ch1[1], bn1_gamma, bn1_beta, conv1_b,
                      jnp.float32(n * 784), 14)

    bt2 = _pick_bt(n, 64)
    p2, st2 = _conv_stage(_k2, p1, s1, t1, bw2, bt=bt2,
                          hout=7, lout=1024, nlanes=1024, yrows=14)
    tot2 = jnp.sum(st2, axis=0)
    ch2 = (tot2[:, 0:448].reshape(2, 7, 64)
           + tot2[:, 512:960].reshape(2, 7, 64)).sum(axis=1)
    s2, t2 = _bn_pair(ch2[0], ch2[1], bn2_gamma, bn2_beta, conv2_b,
                      jnp.float32(n * 196), 7)

    bt3 = _pick_bt(n, 128)
    p3, st3 = _conv_stage(_k3, p2, s2, t2, bw3, bt=bt3,
                          hout=3, lout=768, nlanes=896, yrows=7)
    tot3 = jnp.sum(st3, axis=0)
    ch3 = (tot3[:, 0:512].reshape(2, 4, 128).sum(axis=1)
           + tot3[:, 512:896].reshape(2, 3, 128).sum(axis=1))
    s3, t3 = _bn_pair(ch3[0], ch3[1], bn3_gamma, bn3_beta, conv3_b,
                      jnp.float32(n * 49), 3)

    bt4 = _pick_bt(n, 256)
    out = pl.pallas_call(
        _k4,
        grid=(n // bt4,),
        in_specs=[
            pl.BlockSpec((bt4, 3, 768), lambda i: (i, 0, 0)),
            pl.BlockSpec((1, 384), lambda i: (0, 0)),
            pl.BlockSpec((1, 384), lambda i: (0, 0)),
            pl.BlockSpec(fc1_w_t.shape, lambda i: (0, 0)),
            pl.BlockSpec((1, 256), lambda i: (0, 0)),
            pl.BlockSpec(fc2_w_t.shape, lambda i: (0, 0)),
            pl.BlockSpec((1, 10), lambda i: (0, 0)),
        ],
        out_specs=pl.BlockSpec((bt4, 10), lambda i: (i, 0)),
        out_shape=jax.ShapeDtypeStruct((n, 10), jnp.float32),
        compiler_params=_cparams(),
    )(p3, s3, t3, fc1_w_t, fc1_b.reshape(1, 256), fc2_w_t,
      fc2_b.reshape(1, 10))
    return out
